# Initial kernel scaffold; baseline (speedup 1.0000x reference)
import functools

import jax
import jax.numpy as jnp
from jax import lax
from jax.experimental import pallas as pl
from jax.experimental.pallas import tpu as pltpu
from jax.experimental.pallas import tpu_sc as plsc

N = 10000
E = 320000
D = 128
EPS = 1e-5

NC = 2
NS = 16
K = 128
ROWS_PER_TILE = N // NS
EDGES_PER_TILE = (E // (NC * NS * K)) * K
CHUNKS_MAIN = EDGES_PER_TILE // K
TAIL_BASE = NC * NS * EDGES_PER_TILE
TAIL_CHUNKS = (E - TAIL_BASE) // K

_MESH = plsc.VectorSubcoreMesh(core_axis_name="c", subcore_axis_name="s")


@functools.partial(
    pl.kernel,
    out_type=jax.ShapeDtypeStruct((NC, N, 16), jnp.float32),
    mesh=_MESH,
    scratch_types=[
        pltpu.VMEM_SHARED((N, 16), jnp.float32),
        pltpu.VMEM((K, 16), jnp.float32),
        pltpu.VMEM((K,), jnp.int32),
    ],
)
def _deg_kernel(dst_hbm, zeros_hbm, ones_hbm, out_hbm, dacc, obuf, didx):
    c = lax.axis_index("c")
    s = lax.axis_index("s")
    wid = c * NS + s
    rbase = s * ROWS_PER_TILE
    pltpu.sync_copy(zeros_hbm.at[pl.ds(rbase, ROWS_PER_TILE)],
                    dacc.at[pl.ds(rbase, ROWS_PER_TILE)])
    pltpu.sync_copy(ones_hbm, obuf)
    plsc.subcore_barrier()

    ebase = wid * EDGES_PER_TILE

    def body(j, carry):
        off = ebase + j * K
        pltpu.sync_copy(dst_hbm.at[pl.ds(off, K)], didx)
        pltpu.sync_copy(obuf, dacc.at[didx], add=True)
        return carry

    lax.fori_loop(0, CHUNKS_MAIN, body, 0)

    @pl.when(wid < TAIL_CHUNKS)
    def _():
        off = TAIL_BASE + wid * K
        pltpu.sync_copy(dst_hbm.at[pl.ds(off, K)], didx)
        pltpu.sync_copy(obuf, dacc.at[didx], add=True)

    plsc.subcore_barrier()
    pltpu.sync_copy(dacc.at[pl.ds(rbase, ROWS_PER_TILE)],
                    out_hbm.at[c, pl.ds(rbase, ROWS_PER_TILE)])


@functools.partial(
    pl.kernel,
    out_type=jax.ShapeDtypeStruct((NC, N, D), jnp.float32),
    mesh=_MESH,
    scratch_types=[
        pltpu.VMEM_SHARED((N, D), jnp.float32),
        pltpu.VMEM((K, D), jnp.float32),
        pltpu.VMEM((K,), jnp.int32),
        pltpu.VMEM((K,), jnp.int32),
        pltpu.SemaphoreType.DMA,
    ],
)
def _msg_kernel(y_hbm, src_hbm, dst_hbm, zeros_hbm, out_hbm,
                acc, gbuf, sidx, didx, sem):
    c = lax.axis_index("c")
    s = lax.axis_index("s")
    wid = c * NS + s
    rbase = s * ROWS_PER_TILE

    @pl.when(c == 0)
    def _():
        pltpu.sync_copy(y_hbm.at[pl.ds(rbase, ROWS_PER_TILE)],
                        acc.at[pl.ds(rbase, ROWS_PER_TILE)])

    @pl.when(c == 1)
    def _():
        pltpu.sync_copy(zeros_hbm.at[pl.ds(rbase, ROWS_PER_TILE)],
                        acc.at[pl.ds(rbase, ROWS_PER_TILE)])

    plsc.subcore_barrier()

    ebase = wid * EDGES_PER_TILE

    def body(j, carry):
        off = ebase + j * K
        pltpu.sync_copy(src_hbm.at[pl.ds(off, K)], sidx)
        pltpu.sync_copy(dst_hbm.at[pl.ds(off, K)], didx)
        pltpu.async_copy(y_hbm.at[sidx], gbuf, sem).wait()
        pltpu.sync_copy(gbuf, acc.at[didx], add=True)
        return carry

    lax.fori_loop(0, CHUNKS_MAIN, body, 0)

    @pl.when(wid < TAIL_CHUNKS)
    def _():
        off = TAIL_BASE + wid * K
        pltpu.sync_copy(src_hbm.at[pl.ds(off, K)], sidx)
        pltpu.sync_copy(dst_hbm.at[pl.ds(off, K)], didx)
        pltpu.async_copy(y_hbm.at[sidx], gbuf, sem).wait()
        pltpu.sync_copy(gbuf, acc.at[didx], add=True)

    plsc.subcore_barrier()
    pltpu.sync_copy(acc.at[pl.ds(rbase, ROWS_PER_TILE)],
                    out_hbm.at[c, pl.ds(rbase, ROWS_PER_TILE)])


def _prep_body(degp, x, w0, dinv_out, y_out):
    deg = degp[0, :, 0:1] + degp[1, :, 0:1] + 1.0
    dinv = lax.rsqrt(deg)
    dinv_out[...] = dinv
    y_out[...] = jnp.dot(x[...], w0[...], preferred_element_type=jnp.float32) * dinv


_tc_prep = pl.pallas_call(
    _prep_body,
    out_shape=(
        jax.ShapeDtypeStruct((N, 1), jnp.float32),
        jax.ShapeDtypeStruct((N, D), jnp.float32),
    ),
)


def _bn_relu(z, g, be):
    m = jnp.mean(z, axis=0, keepdims=True)
    v = jnp.mean((z - m) ** 2, axis=0, keepdims=True)
    return jnp.maximum((z - m) * lax.rsqrt(v + EPS) * g + be, 0.0)


def _mid_body(accp, dinv, b, g, be, wn, y_out):
    z = (accp[0] + accp[1]) * dinv[...] + b[...]
    h = _bn_relu(z, g[...], be[...])
    y_out[...] = jnp.dot(h, wn[...], preferred_element_type=jnp.float32) * dinv[...]


_tc_mid = pl.pallas_call(
    _mid_body,
    out_shape=jax.ShapeDtypeStruct((N, D), jnp.float32),
)


def _final_body(accp, dinv, b, g, be, fc1w, fc1b, fc2w, fc2b, out):
    z = (accp[0] + accp[1]) * dinv[...] + b[...]
    h = _bn_relu(z, g[...], be[...])
    t = jnp.maximum(
        jnp.dot(h, fc1w[...], preferred_element_type=jnp.float32) + fc1b[...], 0.0)
    u = jnp.dot(t, fc2w[...], preferred_element_type=jnp.float32) + fc2b[...]
    mx = jnp.max(u, axis=1, keepdims=True)
    lse = mx + jnp.log(jnp.sum(jnp.exp(u - mx), axis=1, keepdims=True))
    out[...] = u - lse


_tc_final = pl.pallas_call(
    _final_body,
    out_shape=jax.ShapeDtypeStruct((N, 2), jnp.float32),
)


def kernel(x, edge_index, W0, b0, W1, b1, W2, b2, bn_g0, bn_b0, bn_g1, bn_b1,
           bn_g2, bn_b2, fc1_W, fc1_b, fc2_W, fc2_b):
    src = edge_index[0].astype(jnp.int32)
    dst = edge_index[1].astype(jnp.int32)
    zerosN = jnp.zeros((N, D), jnp.float32)
    zeros16 = jnp.zeros((N, 16), jnp.float32)
    ones16 = jnp.ones((K, 16), jnp.float32)

    degp = _deg_kernel(dst, zeros16, ones16)
    dinv, y = _tc_prep(degp, x, W0)

    bs = [b0, b1, b2]
    gs = [bn_g0, bn_g1, bn_g2]
    bes = [bn_b0, bn_b1, bn_b2]
    Wn = [W1, W2]
    for l in range(3):
        accp = _msg_kernel(y, src, dst, zerosN)
        b2d = bs[l].reshape(1, D)
        g2d = gs[l].reshape(1, D)
        be2d = bes[l].reshape(1, D)
        if l < 2:
            y = _tc_mid(accp, dinv, b2d, g2d, be2d, Wn[l])
        else:
            out = _tc_final(accp, dinv, b2d, g2d, be2d,
                            fc1_W, fc1_b.reshape(1, 32),
                            fc2_W, fc2_b.reshape(1, 2))
    return out


# speedup vs baseline: 13.6105x; 13.6105x over previous
import functools

import jax
import jax.numpy as jnp
from jax import lax
from jax.experimental import pallas as pl
from jax.experimental.pallas import tpu as pltpu
from jax.experimental.pallas import tpu_sc as plsc

N = 10000
NP = 10240
E = 320000
D = 128
EPS = 1e-5

NC = 2
NS = 16
K = 128
ROWS_PER_TILE = NP // NS
EDGES_PER_TILE = (E // (NC * NS * K)) * K
CHUNKS_MAIN = EDGES_PER_TILE // K
TAIL_BASE = NC * NS * EDGES_PER_TILE
TAIL_CHUNKS = (E - TAIL_BASE) // K

_MESH = plsc.VectorSubcoreMesh(core_axis_name="c", subcore_axis_name="s")


@functools.partial(
    pl.kernel,
    out_type=jax.ShapeDtypeStruct((NC, NP, D), jnp.float32),
    mesh=_MESH,
    scratch_types=[
        pltpu.VMEM_SHARED((NP, D), jnp.float32),
        pltpu.VMEM((K, D), jnp.float32),
        pltpu.VMEM((K,), jnp.int32),
    ],
)
def _deg_kernel(dst_hbm, zeros_hbm, ones_hbm, out_hbm, dacc, obuf, didx):
    c = lax.axis_index("c")
    s = lax.axis_index("s")
    wid = c * NS + s
    rbase = s * ROWS_PER_TILE
    pltpu.sync_copy(zeros_hbm.at[pl.ds(rbase, ROWS_PER_TILE)],
                    dacc.at[pl.ds(rbase, ROWS_PER_TILE)])
    pltpu.sync_copy(ones_hbm, obuf)
    plsc.subcore_barrier()

    ebase = wid * EDGES_PER_TILE

    def body(j, carry):
        off = ebase + j * K
        pltpu.sync_copy(dst_hbm.at[pl.ds(off, K)], didx)
        pltpu.sync_copy(obuf, dacc.at[didx], add=True)
        return carry

    lax.fori_loop(0, CHUNKS_MAIN, body, 0)

    @pl.when(wid < TAIL_CHUNKS)
    def _():
        off = TAIL_BASE + wid * K
        pltpu.sync_copy(dst_hbm.at[pl.ds(off, K)], didx)
        pltpu.sync_copy(obuf, dacc.at[didx], add=True)

    plsc.subcore_barrier()
    pltpu.sync_copy(dacc.at[pl.ds(rbase, ROWS_PER_TILE)],
                    out_hbm.at[c, pl.ds(rbase, ROWS_PER_TILE)])


@functools.partial(
    pl.kernel,
    out_type=jax.ShapeDtypeStruct((NC, NP, D), jnp.float32),
    mesh=_MESH,
    scratch_types=[
        pltpu.VMEM_SHARED((NP, D), jnp.float32),
        pltpu.VMEM((K, D), jnp.float32),
        pltpu.VMEM((K,), jnp.int32),
        pltpu.VMEM((K,), jnp.int32),
        pltpu.SemaphoreType.DMA,
    ],
)
def _msg_kernel(y_hbm, src_hbm, dst_hbm, zeros_hbm, out_hbm,
                acc, gbuf, sidx, didx, sem):
    c = lax.axis_index("c")
    s = lax.axis_index("s")
    wid = c * NS + s
    rbase = s * ROWS_PER_TILE

    @pl.when(c == 0)
    def _():
        pltpu.sync_copy(y_hbm.at[pl.ds(rbase, ROWS_PER_TILE)],
                        acc.at[pl.ds(rbase, ROWS_PER_TILE)])

    @pl.when(c == 1)
    def _():
        pltpu.sync_copy(zeros_hbm.at[pl.ds(rbase, ROWS_PER_TILE)],
                        acc.at[pl.ds(rbase, ROWS_PER_TILE)])

    plsc.subcore_barrier()

    ebase = wid * EDGES_PER_TILE

    def body(j, carry):
        off = ebase + j * K
        pltpu.sync_copy(src_hbm.at[pl.ds(off, K)], sidx)
        pltpu.sync_copy(dst_hbm.at[pl.ds(off, K)], didx)
        pltpu.async_copy(y_hbm.at[sidx], gbuf, sem).wait()
        pltpu.sync_copy(gbuf, acc.at[didx], add=True)
        return carry

    lax.fori_loop(0, CHUNKS_MAIN, body, 0)

    @pl.when(wid < TAIL_CHUNKS)
    def _():
        off = TAIL_BASE + wid * K
        pltpu.sync_copy(src_hbm.at[pl.ds(off, K)], sidx)
        pltpu.sync_copy(dst_hbm.at[pl.ds(off, K)], didx)
        pltpu.async_copy(y_hbm.at[sidx], gbuf, sem).wait()
        pltpu.sync_copy(gbuf, acc.at[didx], add=True)

    plsc.subcore_barrier()
    pltpu.sync_copy(acc.at[pl.ds(rbase, ROWS_PER_TILE)],
                    out_hbm.at[c, pl.ds(rbase, ROWS_PER_TILE)])


def _prep_body(degp, x, w0, dinv_out, y_out):
    deg = degp[0, :N, 0:1] + degp[1, :N, 0:1] + 1.0
    dinv = lax.rsqrt(deg)
    dinv_out[...] = dinv
    y = jnp.dot(x[...], w0[...], preferred_element_type=jnp.float32) * dinv
    y_out[...] = jnp.concatenate([y, jnp.zeros((NP - N, D), jnp.float32)], axis=0)


_tc_prep = pl.pallas_call(
    _prep_body,
    out_shape=(
        jax.ShapeDtypeStruct((N, 1), jnp.float32),
        jax.ShapeDtypeStruct((NP, D), jnp.float32),
    ),
)


def _bn_relu(z, g, be):
    m = jnp.mean(z, axis=0, keepdims=True)
    v = jnp.mean((z - m) ** 2, axis=0, keepdims=True)
    return jnp.maximum((z - m) * lax.rsqrt(v + EPS) * g + be, 0.0)


def _mid_body(accp, dinv, b, g, be, wn, y_out):
    z = (accp[0, :N] + accp[1, :N]) * dinv[...] + b[...]
    h = _bn_relu(z, g[...], be[...])
    y = jnp.dot(h, wn[...], preferred_element_type=jnp.float32) * dinv[...]
    y_out[...] = jnp.concatenate([y, jnp.zeros((NP - N, D), jnp.float32)], axis=0)


_tc_mid = pl.pallas_call(
    _mid_body,
    out_shape=jax.ShapeDtypeStruct((NP, D), jnp.float32),
)


def _final_body(accp, dinv, b, g, be, fc1w, fc1b, fc2w, fc2b, out):
    z = (accp[0, :N] + accp[1, :N]) * dinv[...] + b[...]
    h = _bn_relu(z, g[...], be[...])
    t = jnp.maximum(
        jnp.dot(h, fc1w[...], preferred_element_type=jnp.float32) + fc1b[...], 0.0)
    u = jnp.dot(t, fc2w[...], preferred_element_type=jnp.float32) + fc2b[...]
    mx = jnp.max(u, axis=1, keepdims=True)
    lse = mx + jnp.log(jnp.sum(jnp.exp(u - mx), axis=1, keepdims=True))
    out[...] = u - lse


_tc_final = pl.pallas_call(
    _final_body,
    out_shape=jax.ShapeDtypeStruct((N, 2), jnp.float32),
)


def kernel(x, edge_index, W0, b0, W1, b1, W2, b2, bn_g0, bn_b0, bn_g1, bn_b1,
           bn_g2, bn_b2, fc1_W, fc1_b, fc2_W, fc2_b):
    src = edge_index[0].astype(jnp.int32)
    dst = edge_index[1].astype(jnp.int32)
    zerosN = jnp.zeros((NP, D), jnp.float32)
    onesK = jnp.ones((K, D), jnp.float32)

    degp = _deg_kernel(dst, zerosN, onesK)
    dinv, y = _tc_prep(degp, x, W0)

    bs = [b0, b1, b2]
    gs = [bn_g0, bn_g1, bn_g2]
    bes = [bn_b0, bn_b1, bn_b2]
    Wn = [W1, W2]
    for l in range(3):
        accp = _msg_kernel(y, src, dst, zerosN)
        b2d = bs[l].reshape(1, D)
        g2d = gs[l].reshape(1, D)
        be2d = bes[l].reshape(1, D)
        if l < 2:
            y = _tc_mid(accp, dinv, b2d, g2d, be2d, Wn[l])
        else:
            out = _tc_final(accp, dinv, b2d, g2d, be2d,
                            fc1_W, fc1_b.reshape(1, 32),
                            fc2_W, fc2_b.reshape(1, 2))
    return out


# speedup vs baseline: 24.2982x; 1.7853x over previous
import functools

import jax
import jax.numpy as jnp
from jax import lax
from jax.experimental import pallas as pl
from jax.experimental.pallas import tpu as pltpu
from jax.experimental.pallas import tpu_sc as plsc

N = 10000
NP = 10240
E = 320000
D = 128
EPS = 1e-5

NC = 2
NS = 16
K = 128
ROWS_PER_TILE = NP // NS
CPT = 80
EPAD = NC * NS * CPT * K
NCHUNK = EPAD // K
CB = 16
BLOCKS = CPT // CB
NB = 2

_MESH = plsc.VectorSubcoreMesh(core_axis_name="c", subcore_axis_name="s")


@functools.partial(
    pl.kernel,
    out_type=jax.ShapeDtypeStruct((NC, NP, D), jnp.float32),
    mesh=_MESH,
    scratch_types=[
        pltpu.VMEM_SHARED((NP, D), jnp.float32),
        pltpu.VMEM((K, D), jnp.float32),
        pltpu.VMEM((CB, K), jnp.int32),
        pltpu.SemaphoreType.DMA,
    ],
)
def _deg_kernel(dst_hbm, zeros_hbm, ones_hbm, out_hbm, dacc, obuf, dblk, ssem):
    c = lax.axis_index("c")
    s = lax.axis_index("s")
    wid = c * NS + s
    rbase = s * ROWS_PER_TILE
    pltpu.sync_copy(zeros_hbm.at[pl.ds(rbase, ROWS_PER_TILE)],
                    dacc.at[pl.ds(rbase, ROWS_PER_TILE)])
    pltpu.sync_copy(ones_hbm, obuf)
    plsc.subcore_barrier()

    cbase = wid * CPT

    def block(i, carry):
        pltpu.sync_copy(dst_hbm.at[pl.ds(cbase + i * CB, CB)], dblk)
        sds = []
        for j in range(CB):
            sds.append(pltpu.async_copy(obuf, dacc.at[dblk.at[j]], ssem,
                                        add=True))
        for d in sds:
            d.wait()
        return carry

    lax.fori_loop(0, BLOCKS, block, 0)

    plsc.subcore_barrier()
    pltpu.sync_copy(dacc.at[pl.ds(rbase, ROWS_PER_TILE)],
                    out_hbm.at[c, pl.ds(rbase, ROWS_PER_TILE)])


@functools.partial(
    pl.kernel,
    out_type=jax.ShapeDtypeStruct((NC, NP, D), jnp.float32),
    mesh=_MESH,
    scratch_types=[
        pltpu.VMEM_SHARED((NP, D), jnp.float32),
        pltpu.VMEM((NB, K, D), jnp.float32),
        pltpu.VMEM((CB, K), jnp.int32),
        pltpu.VMEM((CB, K), jnp.int32),
        pltpu.SemaphoreType.DMA((NB,)),
        pltpu.SemaphoreType.DMA((NB,)),
    ],
)
def _msg_kernel(y_hbm, src_hbm, dst_hbm, zeros_hbm, out_hbm,
                acc, gbuf, sblk, dblk, gsem, ssem):
    c = lax.axis_index("c")
    s = lax.axis_index("s")
    wid = c * NS + s
    rbase = s * ROWS_PER_TILE

    @pl.when(c == 0)
    def _():
        pltpu.sync_copy(y_hbm.at[pl.ds(rbase, ROWS_PER_TILE)],
                        acc.at[pl.ds(rbase, ROWS_PER_TILE)])

    @pl.when(c == 1)
    def _():
        pltpu.sync_copy(zeros_hbm.at[pl.ds(rbase, ROWS_PER_TILE)],
                        acc.at[pl.ds(rbase, ROWS_PER_TILE)])

    plsc.subcore_barrier()

    cbase = wid * CPT

    def block(i, carry):
        pltpu.sync_copy(src_hbm.at[pl.ds(cbase + i * CB, CB)], sblk)
        pltpu.sync_copy(dst_hbm.at[pl.ds(cbase + i * CB, CB)], dblk)
        gds = [None] * CB
        sds = [None] * CB
        for j in range(CB):
            b = j % NB
            if j >= NB:
                sds[j - NB].wait()
            gds[j] = pltpu.async_copy(y_hbm.at[sblk.at[j]], gbuf.at[b],
                                      gsem.at[b])
            if j > 0:
                pb = (j - 1) % NB
                gds[j - 1].wait()
                sds[j - 1] = pltpu.async_copy(gbuf.at[pb],
                                              acc.at[dblk.at[j - 1]],
                                              ssem.at[pb], add=True)
        lb = (CB - 1) % NB
        gds[CB - 1].wait()
        sds[CB - 1] = pltpu.async_copy(gbuf.at[lb], acc.at[dblk.at[CB - 1]],
                                       ssem.at[lb], add=True)
        for j in range(CB - NB, CB):
            sds[j].wait()
        return carry

    lax.fori_loop(0, BLOCKS, block, 0)

    plsc.subcore_barrier()
    pltpu.sync_copy(acc.at[pl.ds(rbase, ROWS_PER_TILE)],
                    out_hbm.at[c, pl.ds(rbase, ROWS_PER_TILE)])


def _prep_body(degp, x, w0, dinv_out, y_out):
    deg = degp[0, :N, 0:1] + degp[1, :N, 0:1] + 1.0
    dinv = lax.rsqrt(deg)
    dinv_out[...] = dinv
    y = jnp.dot(x[...], w0[...], preferred_element_type=jnp.float32) * dinv
    y_out[...] = jnp.concatenate([y, jnp.zeros((NP - N, D), jnp.float32)], axis=0)


_tc_prep = pl.pallas_call(
    _prep_body,
    out_shape=(
        jax.ShapeDtypeStruct((N, 1), jnp.float32),
        jax.ShapeDtypeStruct((NP, D), jnp.float32),
    ),
)


def _bn_relu(z, g, be):
    m = jnp.mean(z, axis=0, keepdims=True)
    v = jnp.mean((z - m) ** 2, axis=0, keepdims=True)
    return jnp.maximum((z - m) * lax.rsqrt(v + EPS) * g + be, 0.0)


def _mid_body(accp, dinv, b, g, be, wn, y_out):
    z = (accp[0, :N] + accp[1, :N]) * dinv[...] + b[...]
    h = _bn_relu(z, g[...], be[...])
    y = jnp.dot(h, wn[...], preferred_element_type=jnp.float32) * dinv[...]
    y_out[...] = jnp.concatenate([y, jnp.zeros((NP - N, D), jnp.float32)], axis=0)


_tc_mid = pl.pallas_call(
    _mid_body,
    out_shape=jax.ShapeDtypeStruct((NP, D), jnp.float32),
)


def _final_body(accp, dinv, b, g, be, fc1w, fc1b, fc2w, fc2b, out):
    z = (accp[0, :N] + accp[1, :N]) * dinv[...] + b[...]
    h = _bn_relu(z, g[...], be[...])
    t = jnp.maximum(
        jnp.dot(h, fc1w[...], preferred_element_type=jnp.float32) + fc1b[...], 0.0)
    u = jnp.dot(t, fc2w[...], preferred_element_type=jnp.float32) + fc2b[...]
    mx = jnp.max(u, axis=1, keepdims=True)
    lse = mx + jnp.log(jnp.sum(jnp.exp(u - mx), axis=1, keepdims=True))
    out[...] = u - lse


_tc_final = pl.pallas_call(
    _final_body,
    out_shape=jax.ShapeDtypeStruct((N, 2), jnp.float32),
)


def kernel(x, edge_index, W0, b0, W1, b1, W2, b2, bn_g0, bn_b0, bn_g1, bn_b1,
           bn_g2, bn_b2, fc1_W, fc1_b, fc2_W, fc2_b):
    pad = N + (jnp.arange(EPAD - E, dtype=jnp.int32) % (NP - N))
    src = jnp.concatenate([edge_index[0].astype(jnp.int32), pad]).reshape(NCHUNK, K)
    dst = jnp.concatenate([edge_index[1].astype(jnp.int32), pad]).reshape(NCHUNK, K)
    zerosN = jnp.zeros((NP, D), jnp.float32)
    onesK = jnp.ones((K, D), jnp.float32)

    degp = _deg_kernel(dst, zerosN, onesK)
    dinv, y = _tc_prep(degp, x, W0)

    bs = [b0, b1, b2]
    gs = [bn_g0, bn_g1, bn_g2]
    bes = [bn_b0, bn_b1, bn_b2]
    Wn = [W1, W2]
    for l in range(3):
        accp = _msg_kernel(y, src, dst, zerosN)
        b2d = bs[l].reshape(1, D)
        g2d = gs[l].reshape(1, D)
        be2d = bes[l].reshape(1, D)
        if l < 2:
            y = _tc_mid(accp, dinv, b2d, g2d, be2d, Wn[l])
        else:
            out = _tc_final(accp, dinv, b2d, g2d, be2d,
                            fc1_W, fc1_b.reshape(1, 32),
                            fc2_W, fc2_b.reshape(1, 2))
    return out
